# Initial kernel scaffold; baseline (speedup 1.0000x reference)
import jax
import jax.numpy as jnp
from jax.experimental import pallas as pl


def kernel(assoc, idx_a, idx_b, strength, query_idx, top_k):
    raise NotImplementedError("write your pallas kernel here")


# speedup vs baseline: 4.6612x; 4.6612x over previous
import functools

import jax
import jax.numpy as jnp
from jax import lax
from jax.experimental import pallas as pl
from jax.experimental.pallas import tpu as pltpu
from jax.experimental.pallas import tpu_sc as plsc

M = 8192
B = 16384
NQ = 1024
LR = 0.1
DECAY = 0.001
THRESH = 0.01
K = 16

NC = 2
NS = 16
NW = NC * NS
RPT = 8
NPASS = NQ // (NW * RPT)
GCH = 128

_mesh = lambda: plsc.VectorSubcoreMesh(core_axis_name="c", subcore_axis_name="s")


def _hebbian_scatter_sc(assoc, assoc_flat, idx_a, idx_b, strength, query_idx):
    @functools.partial(
        pl.kernel,
        mesh=_mesh(),
        compiler_params=pltpu.CompilerParams(needs_layout_passes=False),
        out_type=(
            jax.ShapeDtypeStruct((NQ, M), jnp.float32),
            jax.ShapeDtypeStruct((NQ,), jnp.int32),
        ),
        scratch_types=[
            pltpu.VMEM((B,), jnp.int32),
            pltpu.VMEM((B,), jnp.int32),
            pltpu.VMEM((B,), jnp.float32),
            pltpu.VMEM((M,), jnp.int32),
            pltpu.VMEM((NQ,), jnp.int32),
            pltpu.VMEM((RPT, M), jnp.float32),
            pltpu.VMEM((GCH,), jnp.int32),
            pltpu.VMEM((GCH,), jnp.float32),
            pltpu.VMEM((16,), jnp.int32),
            pltpu.VMEM((NQ,), jnp.int32),
            pltpu.SemaphoreType.DMA,
        ],
    )
    def body(assoc_hbm, flat_hbm, ia_hbm, ib_hbm, st_hbm, q_hbm,
             rows_out, rep_out,
             ia_v, ib_v, val_v, slot_v, q_v, rows_v, gidx_v, gcur_v,
             rid_v, rep_v, sem):
        c = lax.axis_index("c")
        s = lax.axis_index("s")
        wid = s * NC + c
        lanes = lax.iota(jnp.int32, 16)

        pltpu.sync_copy(ia_hbm, ia_v)
        pltpu.sync_copy(ib_hbm, ib_v)
        pltpu.sync_copy(st_hbm, val_v)
        pltpu.sync_copy(q_hbm, q_v)

        neg1 = jnp.full((16,), -1, jnp.int32)

        def init_body(i, _):
            slot_v[pl.ds(i * 16, 16)] = neg1
            return 0

        lax.fori_loop(0, M // 16, init_body, 0)

        def qscat_body(j, _):
            q = q_v[pl.ds(j * 16, 16)]
            plsc.store_scatter(slot_v, [q], j * 16 + lanes)
            return 0

        lax.fori_loop(0, NQ // 16, qscat_body, 0)

        def gath_body(g, _):
            def gi(k2, _2):
                t = g * GCH + k2 * 16
                a = ia_v[pl.ds(t, 16)]
                b = ib_v[pl.ds(t, 16)]
                gidx_v[pl.ds(k2 * 16, 16)] = a * M + b
                return 0

            lax.fori_loop(0, GCH // 16, gi, 0)
            pltpu.async_copy(flat_hbm.at[gidx_v], gcur_v, sem).wait()

            def gv(k2, _2):
                t = g * GCH + k2 * 16
                cur = gcur_v[pl.ds(k2 * 16, 16)]
                val_v[pl.ds(t, 16)] = (
                    cur * (1.0 - DECAY) + LR * val_v[pl.ds(t, 16)])
                return 0

            lax.fori_loop(0, GCH // 16, gv, 0)
            return 0

        lax.fori_loop(0, B // GCH, gath_body, 0)

        for p in range(NPASS):
            lo = (p * NW + wid) * RPT
            qsel = lo + jnp.where(lanes < RPT, lanes, 0)
            rid_v[...] = plsc.load_gather(q_v, [qsel])
            pltpu.async_copy(
                assoc_hbm.at[rid_v.at[pl.ds(0, RPT)]], rows_v, sem).wait()

            def mk_side(row_ref, col_ref):
                def sbody(t, _):
                    r = row_ref[pl.ds(t * 16, 16)]
                    cc = col_ref[pl.ds(t * 16, 16)]
                    v = val_v[pl.ds(t * 16, 16)]
                    sl = plsc.load_gather(slot_v, [r])
                    m = (sl >= lo) & (sl < lo + RPT)
                    lr_ = jnp.where(m, sl - lo, 0)
                    c_ = jnp.where(m, cc, 0)
                    plsc.store_scatter(rows_v, [lr_, c_], v, mask=m)
                    return 0

                return sbody

            lax.fori_loop(0, B // 16, mk_side(ia_v, ib_v), 0)
            lax.fori_loop(0, B // 16, mk_side(ib_v, ia_v), 0)
            pltpu.sync_copy(rows_v, rows_out.at[pl.ds(lo, RPT)])

        @pl.when(wid == 0)
        def _():
            def rep_body(j, _):
                q = q_v[pl.ds(j * 16, 16)]
                rep_v[pl.ds(j * 16, 16)] = plsc.load_gather(slot_v, [q])
                return 0

            lax.fori_loop(0, NQ // 16, rep_body, 0)
            pltpu.sync_copy(rep_v, rep_out)

    return body(assoc, assoc_flat, idx_a, idx_b, strength, query_idx)


def _expand_sc(rows_stage, rep):
    per_tile = NQ // NW

    @functools.partial(
        pl.kernel,
        mesh=_mesh(),
        compiler_params=pltpu.CompilerParams(needs_layout_passes=False),
        out_type=jax.ShapeDtypeStruct((NQ, M), jnp.float32),
        scratch_types=[
            pltpu.VMEM((per_tile,), jnp.int32),
            pltpu.VMEM((RPT, M), jnp.float32),
            pltpu.SemaphoreType.DMA,
        ],
    )
    def body(stage_hbm, rep_hbm, out_hbm, rep_v, rows_v, sem):
        c = lax.axis_index("c")
        s = lax.axis_index("s")
        wid = s * NC + c
        base = wid * per_tile
        pltpu.sync_copy(rep_hbm.at[pl.ds(base, per_tile)], rep_v)
        for r in range(per_tile // RPT):
            pltpu.async_copy(
                stage_hbm.at[rep_v.at[pl.ds(r * RPT, RPT)]], rows_v, sem
            ).wait()
            pltpu.sync_copy(rows_v, out_hbm.at[pl.ds(base + r * RPT, RPT)])

    return body(rows_stage, rep)


def _topk_body(rows_ref, val_ref, idx_ref):
    x = rows_ref[...]
    ninf = jnp.float32(-jnp.inf)
    work = jnp.where(x > THRESH, x, ninf)
    colio = lax.broadcasted_iota(jnp.int32, (RPT, M), 1)
    vals, idxs = [], []
    for _ in range(K):
        m = jnp.max(work, axis=1, keepdims=True)
        cand = work == m
        idx = jnp.min(jnp.where(cand, colio, M), axis=1, keepdims=True)
        vals.append(m)
        idxs.append(idx)
        work = jnp.where(colio == idx, ninf, work)
    val_ref[...] = jnp.concatenate(vals, axis=1)[None]
    idx_ref[...] = jnp.concatenate(idxs, axis=1)[None]


def _topk_tc(rows):
    nblk = NQ // RPT
    v3, i3 = pl.pallas_call(
        _topk_body,
        grid=(nblk,),
        in_specs=[pl.BlockSpec((RPT, M), lambda i: (i, 0))],
        out_specs=[
            pl.BlockSpec((1, RPT, K), lambda i: (i, 0, 0)),
            pl.BlockSpec((1, RPT, K), lambda i: (i, 0, 0)),
        ],
        out_shape=[
            jax.ShapeDtypeStruct((nblk, RPT, K), jnp.float32),
            jax.ShapeDtypeStruct((nblk, RPT, K), jnp.int32),
        ],
    )(rows)
    return v3.reshape(NQ, K), i3.reshape(NQ, K)


def kernel(assoc, idx_a, idx_b, strength, query_idx, top_k):
    assoc_flat = assoc.reshape(-1)
    rows_stage, rep = _hebbian_scatter_sc(
        assoc, assoc_flat, idx_a, idx_b, strength, query_idx)
    rows_final = _expand_sc(rows_stage, rep)
    values, indices = _topk_tc(rows_final)
    return values, indices + (top_k - top_k).astype(indices.dtype)


# speedup vs baseline: 7.2502x; 1.5554x over previous
import functools

import jax
import jax.numpy as jnp
from jax import lax
from jax.experimental import pallas as pl
from jax.experimental.pallas import tpu as pltpu
from jax.experimental.pallas import tpu_sc as plsc

M = 8192
B = 16384
NQ = 1024
LR = 0.1
DECAY = 0.001
THRESH = 0.01
K = 16

NC = 2
NS = 16
NW = NC * NS
RPT = 8
NPASS = NQ // (NW * RPT)
GCH = 128

_mesh = lambda: plsc.VectorSubcoreMesh(core_axis_name="c", subcore_axis_name="s")


def _hebbian_scatter_sc(assoc, assoc_flat, idx_a, idx_b, strength, query_idx):
    @functools.partial(
        pl.kernel,
        mesh=_mesh(),
        compiler_params=pltpu.CompilerParams(needs_layout_passes=False),
        out_type=(
            jax.ShapeDtypeStruct((NQ, M), jnp.float32),
            jax.ShapeDtypeStruct((NQ,), jnp.int32),
        ),
        scratch_types=[
            pltpu.VMEM((B,), jnp.int32),
            pltpu.VMEM((B,), jnp.int32),
            pltpu.VMEM((B,), jnp.float32),
            pltpu.VMEM((M,), jnp.int32),
            pltpu.VMEM((NQ,), jnp.int32),
            pltpu.VMEM((RPT, M), jnp.float32),
            pltpu.VMEM((GCH,), jnp.int32),
            pltpu.VMEM((GCH,), jnp.float32),
            pltpu.VMEM((16,), jnp.int32),
            pltpu.VMEM((NQ,), jnp.int32),
            pltpu.SemaphoreType.DMA,
        ],
    )
    def body(assoc_hbm, flat_hbm, ia_hbm, ib_hbm, st_hbm, q_hbm,
             rows_out, rep_out,
             ia_v, ib_v, val_v, slot_v, q_v, rows_v, gidx_v, gcur_v,
             rid_v, rep_v, sem):
        c = lax.axis_index("c")
        s = lax.axis_index("s")
        wid = s * NC + c
        lanes = lax.iota(jnp.int32, 16)

        pltpu.sync_copy(ia_hbm, ia_v)
        pltpu.sync_copy(ib_hbm, ib_v)
        pltpu.sync_copy(st_hbm, val_v)
        pltpu.sync_copy(q_hbm, q_v)

        neg1 = jnp.full((16,), -1, jnp.int32)

        def init_body(i, _):
            slot_v[pl.ds(i * 16, 16)] = neg1
            return 0

        lax.fori_loop(0, M // 16, init_body, 0)

        def qscat_body(j, _):
            q = q_v[pl.ds(j * 16, 16)]
            plsc.store_scatter(slot_v, [q], j * 16 + lanes)
            return 0

        lax.fori_loop(0, NQ // 16, qscat_body, 0)

        def gath_body(g, _):
            def gi(k2, _2):
                t = g * GCH + k2 * 16
                a = ia_v[pl.ds(t, 16)]
                b = ib_v[pl.ds(t, 16)]
                gidx_v[pl.ds(k2 * 16, 16)] = a * M + b
                return 0

            lax.fori_loop(0, GCH // 16, gi, 0)
            pltpu.async_copy(flat_hbm.at[gidx_v], gcur_v, sem).wait()

            def gv(k2, _2):
                t = g * GCH + k2 * 16
                cur = gcur_v[pl.ds(k2 * 16, 16)]
                val_v[pl.ds(t, 16)] = (
                    cur * (1.0 - DECAY) + LR * val_v[pl.ds(t, 16)])
                return 0

            lax.fori_loop(0, GCH // 16, gv, 0)
            return 0

        lax.fori_loop(0, B // GCH, gath_body, 0)

        for p in range(NPASS):
            lo = (p * NW + wid) * RPT
            qsel = lo + jnp.where(lanes < RPT, lanes, 0)
            rid_v[...] = plsc.load_gather(q_v, [qsel])
            pltpu.async_copy(
                assoc_hbm.at[rid_v.at[pl.ds(0, RPT)]], rows_v, sem).wait()

            def mk_side(row_ref, col_ref):
                def sbody(t, _):
                    r = row_ref[pl.ds(t * 16, 16)]
                    cc = col_ref[pl.ds(t * 16, 16)]
                    v = val_v[pl.ds(t * 16, 16)]
                    sl = plsc.load_gather(slot_v, [r])
                    m = (sl >= lo) & (sl < lo + RPT)
                    lr_ = jnp.where(m, sl - lo, 0)
                    c_ = jnp.where(m, cc, 0)
                    plsc.store_scatter(rows_v, [lr_, c_], v, mask=m)
                    return 0

                return sbody

            lax.fori_loop(0, B // 16, mk_side(ia_v, ib_v), 0)
            lax.fori_loop(0, B // 16, mk_side(ib_v, ia_v), 0)
            pltpu.sync_copy(rows_v, rows_out.at[pl.ds(lo, RPT)])

        @pl.when(wid == 0)
        def _():
            def rep_body(j, _):
                q = q_v[pl.ds(j * 16, 16)]
                rep_v[pl.ds(j * 16, 16)] = plsc.load_gather(slot_v, [q])
                return 0

            lax.fori_loop(0, NQ // 16, rep_body, 0)
            pltpu.sync_copy(rep_v, rep_out)

    return body(assoc, assoc_flat, idx_a, idx_b, strength, query_idx)


def _expand_sc(rows_stage, rep):
    per_tile = NQ // NW

    @functools.partial(
        pl.kernel,
        mesh=_mesh(),
        compiler_params=pltpu.CompilerParams(needs_layout_passes=False),
        out_type=jax.ShapeDtypeStruct((NQ, M), jnp.float32),
        scratch_types=[
            pltpu.VMEM((per_tile,), jnp.int32),
            pltpu.VMEM((RPT, M), jnp.float32),
            pltpu.SemaphoreType.DMA,
        ],
    )
    def body(stage_hbm, rep_hbm, out_hbm, rep_v, rows_v, sem):
        c = lax.axis_index("c")
        s = lax.axis_index("s")
        wid = s * NC + c
        base = wid * per_tile
        pltpu.sync_copy(rep_hbm.at[pl.ds(base, per_tile)], rep_v)
        for r in range(per_tile // RPT):
            pltpu.async_copy(
                stage_hbm.at[rep_v.at[pl.ds(r * RPT, RPT)]], rows_v, sem
            ).wait()
            pltpu.sync_copy(rows_v, out_hbm.at[pl.ds(base + r * RPT, RPT)])

    return body(rows_stage, rep)


RPT_TC = 32


def _topk_body(rows_ref, val_ref, idx_ref):
    x = rows_ref[...]
    ninf = jnp.float32(-jnp.inf)
    work = jnp.where(x > THRESH, x, ninf)
    colio = lax.broadcasted_iota(jnp.int32, (RPT_TC, M), 1)
    vals, idxs = [], []
    for _ in range(K):
        m = jnp.max(work, axis=1, keepdims=True)
        cand = work == m
        idx = jnp.min(jnp.where(cand, colio, M), axis=1, keepdims=True)
        vals.append(m)
        idxs.append(idx)
        work = jnp.where(colio == idx, ninf, work)
    val_ref[...] = jnp.concatenate(vals, axis=1)[None]
    idx_ref[...] = jnp.concatenate(idxs, axis=1)[None]


def _topk_tc(rows):
    nblk = NQ // RPT_TC
    v3, i3 = pl.pallas_call(
        _topk_body,
        grid=(nblk,),
        in_specs=[pl.BlockSpec((RPT_TC, M), lambda i: (i, 0))],
        out_specs=[
            pl.BlockSpec((1, RPT_TC, K), lambda i: (i, 0, 0)),
            pl.BlockSpec((1, RPT_TC, K), lambda i: (i, 0, 0)),
        ],
        out_shape=[
            jax.ShapeDtypeStruct((nblk, RPT_TC, K), jnp.float32),
            jax.ShapeDtypeStruct((nblk, RPT_TC, K), jnp.int32),
        ],
    )(rows)
    return v3.reshape(NQ, K), i3.reshape(NQ, K)


def kernel(assoc, idx_a, idx_b, strength, query_idx, top_k):
    assoc_flat = assoc.reshape(-1)
    rows_stage, rep = _hebbian_scatter_sc(
        assoc, assoc_flat, idx_a, idx_b, strength, query_idx)
    rows_final = _expand_sc(rows_stage, rep)
    values, indices = _topk_tc(rows_final)
    return values, indices + (top_k - top_k).astype(indices.dtype)


# speedup vs baseline: 8.6116x; 1.1878x over previous
import functools

import jax
import jax.numpy as jnp
from jax import lax
from jax.experimental import pallas as pl
from jax.experimental.pallas import tpu as pltpu
from jax.experimental.pallas import tpu_sc as plsc

M = 8192
B = 16384
NQ = 1024
LR = 0.1
DECAY = 0.001
THRESH = 0.01
K = 16

NC = 2
NS = 16
NW = NC * NS
RPT = 8
NPASS = NQ // (NW * RPT)
GCH = 128

_mesh = lambda: plsc.VectorSubcoreMesh(core_axis_name="c", subcore_axis_name="s")


def _hebbian_scatter_sc(assoc, assoc_flat, idx_a, idx_b, strength, query_idx):
    @functools.partial(
        pl.kernel,
        mesh=_mesh(),
        compiler_params=pltpu.CompilerParams(needs_layout_passes=False),
        out_type=(
            jax.ShapeDtypeStruct((NQ, M), jnp.float32),
            jax.ShapeDtypeStruct((NQ,), jnp.int32),
        ),
        scratch_types=[
            pltpu.VMEM((B,), jnp.int32),
            pltpu.VMEM((B,), jnp.int32),
            pltpu.VMEM((B,), jnp.float32),
            pltpu.VMEM((M,), jnp.int32),
            pltpu.VMEM((NQ,), jnp.int32),
            pltpu.VMEM((RPT, M), jnp.float32),
            pltpu.VMEM((GCH,), jnp.int32),
            pltpu.VMEM((GCH,), jnp.float32),
            pltpu.VMEM((16,), jnp.int32),
            pltpu.VMEM((NQ,), jnp.int32),
            pltpu.SemaphoreType.DMA,
        ],
    )
    def body(assoc_hbm, flat_hbm, ia_hbm, ib_hbm, st_hbm, q_hbm,
             rows_out, rep_out,
             ia_v, ib_v, val_v, slot_v, q_v, rows_v, gidx_v, gcur_v,
             rid_v, rep_v, sem):
        c = lax.axis_index("c")
        s = lax.axis_index("s")
        wid = s * NC + c
        lanes = lax.iota(jnp.int32, 16)

        pltpu.sync_copy(ia_hbm, ia_v)
        pltpu.sync_copy(ib_hbm, ib_v)
        pltpu.sync_copy(st_hbm, val_v)
        pltpu.sync_copy(q_hbm, q_v)

        neg1 = jnp.full((16,), -1, jnp.int32)

        @plsc.parallel_loop(0, M // 16, unroll=8)
        def _(i):
            slot_v[pl.ds(i * 16, 16)] = neg1

        def qscat_body(j, _):
            q = q_v[pl.ds(j * 16, 16)]
            plsc.store_scatter(slot_v, [q], j * 16 + lanes)
            return 0

        lax.fori_loop(0, NQ // 16, qscat_body, 0)

        def gath_body(g, _):
            def gi(k2, _2):
                t = g * GCH + k2 * 16
                a = ia_v[pl.ds(t, 16)]
                b = ib_v[pl.ds(t, 16)]
                gidx_v[pl.ds(k2 * 16, 16)] = a * M + b
                return 0

            lax.fori_loop(0, GCH // 16, gi, 0)
            pltpu.async_copy(flat_hbm.at[gidx_v], gcur_v, sem).wait()

            def gv(k2, _2):
                t = g * GCH + k2 * 16
                cur = gcur_v[pl.ds(k2 * 16, 16)]
                val_v[pl.ds(t, 16)] = (
                    cur * (1.0 - DECAY) + LR * val_v[pl.ds(t, 16)])
                return 0

            lax.fori_loop(0, GCH // 16, gv, 0)
            return 0

        lax.fori_loop(0, B // GCH, gath_body, 0)

        for p in range(NPASS):
            lo = (p * NW + wid) * RPT
            qsel = lo + jnp.where(lanes < RPT, lanes, 0)
            rid_v[...] = plsc.load_gather(q_v, [qsel])
            pltpu.async_copy(
                assoc_hbm.at[rid_v.at[pl.ds(0, RPT)]], rows_v, sem).wait()

            def mk_side(row_ref, col_ref):
                def sbody(t):
                    r = row_ref[pl.ds(t * 16, 16)]
                    cc = col_ref[pl.ds(t * 16, 16)]
                    v = val_v[pl.ds(t * 16, 16)]
                    sl = plsc.load_gather(slot_v, [r])
                    m = (sl >= lo) & (sl < lo + RPT)
                    lr_ = jnp.where(m, sl - lo, 0)
                    c_ = jnp.where(m, cc, 0)
                    plsc.store_scatter(rows_v, [lr_, c_], v, mask=m)

                return sbody

            plsc.parallel_loop(0, B // 16, unroll=8)(mk_side(ia_v, ib_v))
            plsc.parallel_loop(0, B // 16, unroll=8)(mk_side(ib_v, ia_v))
            pltpu.sync_copy(rows_v, rows_out.at[pl.ds(lo, RPT)])

        @pl.when(wid == 0)
        def _():
            def rep_body(j, _):
                q = q_v[pl.ds(j * 16, 16)]
                rep_v[pl.ds(j * 16, 16)] = plsc.load_gather(slot_v, [q])
                return 0

            lax.fori_loop(0, NQ // 16, rep_body, 0)
            pltpu.sync_copy(rep_v, rep_out)

    return body(assoc, assoc_flat, idx_a, idx_b, strength, query_idx)


def _expand_sc(rows_stage, rep):
    per_tile = NQ // NW

    @functools.partial(
        pl.kernel,
        mesh=_mesh(),
        compiler_params=pltpu.CompilerParams(needs_layout_passes=False),
        out_type=jax.ShapeDtypeStruct((NQ, M), jnp.float32),
        scratch_types=[
            pltpu.VMEM((per_tile,), jnp.int32),
            pltpu.VMEM((RPT, M), jnp.float32),
            pltpu.SemaphoreType.DMA,
        ],
    )
    def body(stage_hbm, rep_hbm, out_hbm, rep_v, rows_v, sem):
        c = lax.axis_index("c")
        s = lax.axis_index("s")
        wid = s * NC + c
        base = wid * per_tile
        pltpu.sync_copy(rep_hbm.at[pl.ds(base, per_tile)], rep_v)
        for r in range(per_tile // RPT):
            pltpu.async_copy(
                stage_hbm.at[rep_v.at[pl.ds(r * RPT, RPT)]], rows_v, sem
            ).wait()
            pltpu.sync_copy(rows_v, out_hbm.at[pl.ds(base + r * RPT, RPT)])

    return body(rows_stage, rep)


RPT_TC = 32


def _topk_body(rows_ref, val_ref, idx_ref):
    x = rows_ref[...]
    ninf = jnp.float32(-jnp.inf)
    work = jnp.where(x > THRESH, x, ninf)
    colio = lax.broadcasted_iota(jnp.int32, (RPT_TC, M), 1)
    vals, idxs = [], []
    for _ in range(K):
        m = jnp.max(work, axis=1, keepdims=True)
        cand = work == m
        idx = jnp.min(jnp.where(cand, colio, M), axis=1, keepdims=True)
        vals.append(m)
        idxs.append(idx)
        work = jnp.where(colio == idx, ninf, work)
    val_ref[...] = jnp.concatenate(vals, axis=1)[None]
    idx_ref[...] = jnp.concatenate(idxs, axis=1)[None]


def _topk_tc(rows):
    nblk = NQ // RPT_TC
    v3, i3 = pl.pallas_call(
        _topk_body,
        grid=(nblk,),
        in_specs=[pl.BlockSpec((RPT_TC, M), lambda i: (i, 0))],
        out_specs=[
            pl.BlockSpec((1, RPT_TC, K), lambda i: (i, 0, 0)),
            pl.BlockSpec((1, RPT_TC, K), lambda i: (i, 0, 0)),
        ],
        out_shape=[
            jax.ShapeDtypeStruct((nblk, RPT_TC, K), jnp.float32),
            jax.ShapeDtypeStruct((nblk, RPT_TC, K), jnp.int32),
        ],
    )(rows)
    return v3.reshape(NQ, K), i3.reshape(NQ, K)


def kernel(assoc, idx_a, idx_b, strength, query_idx, top_k):
    assoc_flat = assoc.reshape(-1)
    rows_stage, rep = _hebbian_scatter_sc(
        assoc, assoc_flat, idx_a, idx_b, strength, query_idx)
    rows_final = _expand_sc(rows_stage, rep)
    values, indices = _topk_tc(rows_final)
    return values, indices + (top_k - top_k).astype(indices.dtype)


# speedup vs baseline: 10.2723x; 1.1928x over previous
import functools

import jax
import jax.numpy as jnp
from jax import lax
from jax.experimental import pallas as pl
from jax.experimental.pallas import tpu as pltpu
from jax.experimental.pallas import tpu_sc as plsc

M = 8192
B = 16384
NQ = 1024
LR = 0.1
DECAY = 0.001
THRESH = 0.01
K = 16

NC = 2
NS = 16
NW = NC * NS
RPT = 8
NPASS = NQ // (NW * RPT)
GCH = 128
NBUF = 4

_mesh = lambda: plsc.VectorSubcoreMesh(core_axis_name="c", subcore_axis_name="s")


def _hebbian_scatter_sc(assoc, assoc_flat, idx_a, idx_b, strength, query_idx):
    @functools.partial(
        pl.kernel,
        mesh=_mesh(),
        compiler_params=pltpu.CompilerParams(needs_layout_passes=False),
        out_type=(
            jax.ShapeDtypeStruct((NQ, M), jnp.float32),
            jax.ShapeDtypeStruct((NQ,), jnp.int32),
        ),
        scratch_types=[
            pltpu.VMEM((B,), jnp.int32),
            pltpu.VMEM((B,), jnp.int32),
            pltpu.VMEM((B,), jnp.float32),
            pltpu.VMEM((M,), jnp.int32),
            pltpu.VMEM((NQ,), jnp.int32),
            pltpu.VMEM((RPT, M), jnp.float32),
            pltpu.VMEM((NBUF * GCH,), jnp.int32),
            pltpu.VMEM((NBUF * GCH,), jnp.float32),
            pltpu.VMEM((16,), jnp.int32),
            pltpu.VMEM((NQ,), jnp.int32),
            pltpu.SemaphoreType.DMA,
        ],
    )
    def body(assoc_hbm, flat_hbm, ia_hbm, ib_hbm, st_hbm, q_hbm,
             rows_out, rep_out,
             ia_v, ib_v, val_v, slot_v, q_v, rows_v, gidx_v, gcur_v,
             rid_v, rep_v, sem):
        c = lax.axis_index("c")
        s = lax.axis_index("s")
        wid = s * NC + c
        lanes = lax.iota(jnp.int32, 16)

        pltpu.sync_copy(ia_hbm, ia_v)
        pltpu.sync_copy(ib_hbm, ib_v)
        pltpu.sync_copy(st_hbm, val_v)
        pltpu.sync_copy(q_hbm, q_v)

        neg1 = jnp.full((16,), -1, jnp.int32)

        @plsc.parallel_loop(0, M // 16, unroll=8)
        def _(i):
            slot_v[pl.ds(i * 16, 16)] = neg1

        def qscat_body(j, _):
            q = q_v[pl.ds(j * 16, 16)]
            plsc.store_scatter(slot_v, [q], j * 16 + lanes)
            return 0

        lax.fori_loop(0, NQ // 16, qscat_body, 0)

        ngc = B // GCH
        handles = [None] * NBUF

        def fill_idx(g, buf):
            for k2 in range(GCH // 16):
                t = g * GCH + k2 * 16
                a = ia_v[pl.ds(t, 16)]
                b = ib_v[pl.ds(t, 16)]
                gidx_v[pl.ds(buf * GCH + k2 * 16, 16)] = a * M + b

        def use_cur(g, buf):
            for k2 in range(GCH // 16):
                t = g * GCH + k2 * 16
                cur = gcur_v[pl.ds(buf * GCH + k2 * 16, 16)]
                val_v[pl.ds(t, 16)] = (
                    cur * (1.0 - DECAY) + LR * val_v[pl.ds(t, 16)])

        for g in range(ngc + NBUF):
            buf = g % NBUF
            if g >= NBUF:
                handles[buf].wait()
                use_cur(g - NBUF, buf)
            if g < ngc:
                fill_idx(g, buf)
                handles[buf] = pltpu.async_copy(
                    flat_hbm.at[gidx_v.at[pl.ds(buf * GCH, GCH)]],
                    gcur_v.at[pl.ds(buf * GCH, GCH)],
                    sem,
                )

        for p in range(NPASS):
            lo = (p * NW + wid) * RPT
            qsel = lo + jnp.where(lanes < RPT, lanes, 0)
            rid_v[...] = plsc.load_gather(q_v, [qsel])
            pltpu.async_copy(
                assoc_hbm.at[rid_v.at[pl.ds(0, RPT)]], rows_v, sem).wait()

            def mk_side(row_ref, col_ref):
                def sbody(t):
                    r = row_ref[pl.ds(t * 16, 16)]
                    cc = col_ref[pl.ds(t * 16, 16)]
                    v = val_v[pl.ds(t * 16, 16)]
                    sl = plsc.load_gather(slot_v, [r])
                    m = (sl >= lo) & (sl < lo + RPT)
                    lr_ = jnp.where(m, sl - lo, 0)
                    c_ = jnp.where(m, cc, 0)
                    plsc.store_scatter(rows_v, [lr_, c_], v, mask=m)

                return sbody

            plsc.parallel_loop(0, B // 16, unroll=8)(mk_side(ia_v, ib_v))
            plsc.parallel_loop(0, B // 16, unroll=8)(mk_side(ib_v, ia_v))
            pltpu.sync_copy(rows_v, rows_out.at[pl.ds(lo, RPT)])

        @pl.when(wid == 0)
        def _():
            def rep_body(j, _):
                q = q_v[pl.ds(j * 16, 16)]
                rep_v[pl.ds(j * 16, 16)] = plsc.load_gather(slot_v, [q])
                return 0

            lax.fori_loop(0, NQ // 16, rep_body, 0)
            pltpu.sync_copy(rep_v, rep_out)

    return body(assoc, assoc_flat, idx_a, idx_b, strength, query_idx)


def _expand_sc(rows_stage, rep):
    per_tile = NQ // NW

    @functools.partial(
        pl.kernel,
        mesh=_mesh(),
        compiler_params=pltpu.CompilerParams(needs_layout_passes=False),
        out_type=jax.ShapeDtypeStruct((NQ, M), jnp.float32),
        scratch_types=[
            pltpu.VMEM((per_tile,), jnp.int32),
            pltpu.VMEM((RPT, M), jnp.float32),
            pltpu.SemaphoreType.DMA,
        ],
    )
    def body(stage_hbm, rep_hbm, out_hbm, rep_v, rows_v, sem):
        c = lax.axis_index("c")
        s = lax.axis_index("s")
        wid = s * NC + c
        base = wid * per_tile
        pltpu.sync_copy(rep_hbm.at[pl.ds(base, per_tile)], rep_v)
        for r in range(per_tile // RPT):
            pltpu.async_copy(
                stage_hbm.at[rep_v.at[pl.ds(r * RPT, RPT)]], rows_v, sem
            ).wait()
            pltpu.sync_copy(rows_v, out_hbm.at[pl.ds(base + r * RPT, RPT)])

    return body(rows_stage, rep)


RPT_TC = 64


def _topk_body(rows_ref, val_ref, idx_ref):
    x = rows_ref[...]
    ninf = jnp.float32(-jnp.inf)
    work = jnp.where(x > THRESH, x, ninf)
    colio = lax.broadcasted_iota(jnp.int32, (RPT_TC, M), 1)
    vals, idxs = [], []
    for _ in range(K):
        m = jnp.max(work, axis=1, keepdims=True)
        cand = work == m
        idx = jnp.min(jnp.where(cand, colio, M), axis=1, keepdims=True)
        vals.append(m)
        idxs.append(idx)
        work = jnp.where(colio == idx, ninf, work)
    val_ref[...] = jnp.concatenate(vals, axis=1)[None]
    idx_ref[...] = jnp.concatenate(idxs, axis=1)[None]


def _topk_tc(rows):
    nblk = NQ // RPT_TC
    v3, i3 = pl.pallas_call(
        _topk_body,
        grid=(nblk,),
        in_specs=[pl.BlockSpec((RPT_TC, M), lambda i: (i, 0))],
        out_specs=[
            pl.BlockSpec((1, RPT_TC, K), lambda i: (i, 0, 0)),
            pl.BlockSpec((1, RPT_TC, K), lambda i: (i, 0, 0)),
        ],
        out_shape=[
            jax.ShapeDtypeStruct((nblk, RPT_TC, K), jnp.float32),
            jax.ShapeDtypeStruct((nblk, RPT_TC, K), jnp.int32),
        ],
    )(rows)
    return v3.reshape(NQ, K), i3.reshape(NQ, K)


def kernel(assoc, idx_a, idx_b, strength, query_idx, top_k):
    assoc_flat = assoc.reshape(-1)
    rows_stage, rep = _hebbian_scatter_sc(
        assoc, assoc_flat, idx_a, idx_b, strength, query_idx)
    rows_final = _expand_sc(rows_stage, rep)
    values, indices = _topk_tc(rows_final)
    return values, indices + (top_k - top_k).astype(indices.dtype)
